# baseline (speedup 1.0000x reference)
import functools

import jax
import jax.numpy as jnp
from jax import lax
from jax.experimental import pallas as pl
from jax.experimental.pallas import tpu as pltpu
from jax.experimental.pallas import tpu_sc as plsc

B, C, N = 8, 128, 2048
K = 20
RB = 1024
NW = 32
RCH = 4
SCH = 16
NGA = SCH // RCH
CHUNK = 1


def _topk_uv_body(xf_ref, xs_ref, w1_ref, wd_ref, b_ref, idx_ref, u_ref, v_ref):
    xb = xf_ref[0]
    xs = xs_ref[0]
    xx = jnp.sum(xb * xb, axis=0, keepdims=True)
    inner = lax.dot_general(xs, xb, (((0,), (0,)), ((), ())),
                            preferred_element_type=jnp.float32)
    scores = xx - 2.0 * inner
    iota_m = lax.broadcasted_iota(jnp.int32, (RB, N), 1)
    iota_k = lax.broadcasted_iota(jnp.int32, (RB, K), 1)
    rmin = jnp.min(scores, axis=1, keepdims=True)
    rmax = jnp.max(scores, axis=1, keepdims=True)
    scale = 1048575.0 / (rmax - rmin + 1e-20)
    q = ((scores - rmin) * scale).astype(jnp.int32)
    key = (q << 11) | iota_m
    idxs = jnp.zeros((RB, K), jnp.int32)
    for k in range(K):
        m = jnp.min(key, axis=1, keepdims=True)
        key = jnp.where(key == m, jnp.int32(2**31 - 1), key)
        idxs = jnp.where(iota_k == k, m & 2047, idxs)
    idx_ref[...] = idxs
    u_ref[...] = lax.dot_general(xs, w1_ref[...], (((0,), (1,)), ((), ())),
                                 preferred_element_type=jnp.float32)
    v_ref[...] = lax.dot_general(xs, wd_ref[...], (((0,), (1,)), ((), ())),
                                 preferred_element_type=jnp.float32) + b_ref[...]


def _topk_uv(xb, w1, wd, b2):
    return pl.pallas_call(
        _topk_uv_body,
        grid=(N // RB,),
        in_specs=[
            pl.BlockSpec((1, C, N), lambda j: (0, 0, 0)),
            pl.BlockSpec((1, C, RB), lambda j: (0, 0, j)),
            pl.BlockSpec((C, C), lambda j: (0, 0)),
            pl.BlockSpec((C, C), lambda j: (0, 0)),
            pl.BlockSpec((1, C), lambda j: (0, 0)),
        ],
        out_specs=[
            pl.BlockSpec((RB, K), lambda j: (j, 0)),
            pl.BlockSpec((RB, C), lambda j: (j, 0)),
            pl.BlockSpec((RB, C), lambda j: (j, 0)),
        ],
        out_shape=[
            jax.ShapeDtypeStruct((N, K), jnp.int32),
            jax.ShapeDtypeStruct((N, C), jnp.float32),
            jax.ShapeDtypeStruct((N, C), jnp.float32),
        ],
    )(xb, xb, w1, wd, b2)


def _make_sc_aggregate(rows=N):
    nsc_ch = rows // (NW * SCH)
    mesh = plsc.VectorSubcoreMesh(core_axis_name="c", subcore_axis_name="s")

    @functools.partial(
        pl.kernel,
        mesh=mesh,
        out_type=jax.ShapeDtypeStruct((rows, C), jnp.float32),
        scratch_types=[
            pltpu.VMEM((2, NGA, RCH * K), jnp.int32),
            pltpu.VMEM((2, SCH * K, C), jnp.float32),
            pltpu.VMEM((2, SCH, C), jnp.float32),
            pltpu.VMEM((2, SCH, C), jnp.float32),
            pltpu.SemaphoreType.DMA,
            pltpu.SemaphoreType.DMA,
            pltpu.SemaphoreType.DMA,
            pltpu.SemaphoreType.DMA,
            pltpu.SemaphoreType.DMA,
            pltpu.SemaphoreType.DMA,
        ],
    )
    def agg(u_hbm, v_hbm, idx_hbm, out_hbm, idx_v, ug_v, v_v, o_v,
            si0, si1, sg0, sg1, so0, so1):
        wid = lax.axis_index("s") * 2 + lax.axis_index("c")
        base = wid * nsc_ch
        sem_i, sem_g, sem_o = (si0, si1), (sg0, sg1), (so0, so1)

        def fetch_idx(sc, p):
            g0 = pl.multiple_of((base + sc) * NGA, NGA)
            pltpu.async_copy(idx_hbm.at[pl.ds(g0, NGA)], idx_v.at[p],
                             sem_i[p])

        def wait_idx(p):
            pltpu.make_async_copy(idx_hbm.at[pl.ds(0, NGA)], idx_v.at[p],
                                  sem_i[p]).wait()

        def issue_main(sc, p):
            row0 = pl.multiple_of((base + sc) * SCH, SCH)
            for g in range(NGA):
                pltpu.async_copy(u_hbm.at[idx_v.at[p, g]],
                                 ug_v.at[p, pl.ds(g * RCH * K, RCH * K)],
                                 sem_g[p])
            pltpu.async_copy(v_hbm.at[pl.ds(row0, SCH)], v_v.at[p], sem_g[p])

        def wait_main(p):
            pltpu.make_async_copy(u_hbm.at[pl.ds(0, SCH * K)], ug_v.at[p],
                                  sem_g[p]).wait()
            pltpu.make_async_copy(v_hbm.at[pl.ds(0, SCH)], v_v.at[p],
                                  sem_g[p]).wait()

        def wait_out(p):
            pltpu.make_async_copy(o_v.at[p], out_hbm.at[pl.ds(0, SCH)],
                                  sem_o[p]).wait()

        def compute_store(sc, p):
            def row_body(r, carry, p=p):
                accs = [v_v[p, r, pl.ds(j * 16, 16)] for j in range(C // 16)]
                vvs = list(accs)
                for k in range(K):
                    for j in range(C // 16):
                        s = ug_v[p, r * K + k, pl.ds(j * 16, 16)] + vvs[j]
                        h = jnp.maximum(s, 0.2 * s)
                        accs[j] = h if k == 0 else accs[j] + h
                for j in range(C // 16):
                    o_v[p, r, pl.ds(j * 16, 16)] = accs[j] * (1.0 / K)
                return carry

            lax.fori_loop(0, SCH, row_body, 0)
            row0 = pl.multiple_of((base + sc) * SCH, SCH)
            pltpu.async_copy(o_v.at[p], out_hbm.at[pl.ds(row0, SCH)],
                             sem_o[p])

        fetch_idx(0, 0)
        wait_idx(0)
        issue_main(0, 0)
        if nsc_ch > 1:
            fetch_idx(1, 1)

        def pair_body(ii, carry):
            for pp in range(2):
                sc = ii * 2 + pp

                @pl.when(sc + 1 < nsc_ch)
                def _(pp=pp, sc=sc):
                    wait_idx(1 - pp)
                    issue_main(sc + 1, 1 - pp)

                wait_main(pp)

                @pl.when(sc + 2 < nsc_ch)
                def _(pp=pp, sc=sc):
                    fetch_idx(sc + 2, pp)

                @pl.when(sc >= 2)
                def _(pp=pp):
                    wait_out(pp)

                compute_store(sc, pp)
            return carry

        lax.fori_loop(0, nsc_ch // 2, pair_body, 0)
        wait_out(0)
        if nsc_ch > 1:
            wait_out(1)

    return agg


def kernel(x, W, b):
    w1 = W[:, :C]
    wd = W[:, C:] - w1
    b2 = b.reshape(1, C)
    outs = []
    for c0 in range(0, B, CHUNK):
        us, vs, ids = [], [], []
        for bi in range(c0, c0 + CHUNK):
            xb = lax.slice_in_dim(x, bi, bi + 1, axis=0)
            idx, u, v = _topk_uv(xb, w1, wd, b2)
            us.append(u); vs.append(v); ids.append(idx + (bi - c0) * N)
        u_t = jnp.concatenate(us) if CHUNK > 1 else us[0]
        v_t = jnp.concatenate(vs) if CHUNK > 1 else vs[0]
        i_t = jnp.concatenate(ids) if CHUNK > 1 else ids[0]
        outs.append(_sc_agg(u_t, v_t,
                            i_t.reshape((CHUNK * N * K) // (RCH * K),
                                        RCH * K)))
    o = jnp.concatenate(outs) if len(outs) > 1 else outs[0]
    return o.reshape(B, N, C).transpose(0, 2, 1)


_sc_agg = _make_sc_aggregate(CHUNK * N)


# speedup vs baseline: 1.2753x; 1.2753x over previous
import functools

import jax
import jax.numpy as jnp
from jax import lax
from jax.experimental import pallas as pl
from jax.experimental.pallas import tpu as pltpu
from jax.experimental.pallas import tpu_sc as plsc

B, C, N = 8, 128, 2048
K = 20
RB = 512
NW = 32
RCH = 4
SCH = 16
NGA = SCH // RCH
CHUNK = 1


def _topk_uv_body(xf_ref, xs_ref, w1_ref, wd_ref, b_ref, idx_ref, u_ref, v_ref):
    xb = xf_ref[0]
    xs = xs_ref[0]
    xx = jnp.sum(xb * xb, axis=0, keepdims=True)
    inner = lax.dot_general(xs, xb, (((0,), (0,)), ((), ())),
                            preferred_element_type=jnp.float32)
    scores = xx - 2.0 * inner
    iota_m = lax.broadcasted_iota(jnp.int32, (RB, N), 1)
    iota_k = lax.broadcasted_iota(jnp.int32, (RB, K), 1)
    rmin = jnp.min(scores, axis=1, keepdims=True)
    rmax = jnp.max(scores, axis=1, keepdims=True)
    scale = 1048575.0 / (rmax - rmin + 1e-20)
    q = ((scores - rmin) * scale).astype(jnp.int32)
    key = (q << 11) | iota_m
    idxs = jnp.zeros((RB, K), jnp.int32)
    for k in range(K):
        m = jnp.min(key, axis=1, keepdims=True)
        key = jnp.where(key == m, jnp.int32(2**31 - 1), key)
        idxs = jnp.where(iota_k == k, m & 2047, idxs)
    idx_ref[...] = idxs
    u_ref[...] = lax.dot_general(xs, w1_ref[...], (((0,), (1,)), ((), ())),
                                 preferred_element_type=jnp.float32)
    v_ref[...] = lax.dot_general(xs, wd_ref[...], (((0,), (1,)), ((), ())),
                                 preferred_element_type=jnp.float32) + b_ref[...]


def _topk_uv(xb, w1, wd, b2):
    return pl.pallas_call(
        _topk_uv_body,
        grid=(N // RB,),
        in_specs=[
            pl.BlockSpec((1, C, N), lambda j: (0, 0, 0)),
            pl.BlockSpec((1, C, RB), lambda j: (0, 0, j)),
            pl.BlockSpec((C, C), lambda j: (0, 0)),
            pl.BlockSpec((C, C), lambda j: (0, 0)),
            pl.BlockSpec((1, C), lambda j: (0, 0)),
        ],
        out_specs=[
            pl.BlockSpec((RB, K), lambda j: (j, 0)),
            pl.BlockSpec((RB, C), lambda j: (j, 0)),
            pl.BlockSpec((RB, C), lambda j: (j, 0)),
        ],
        out_shape=[
            jax.ShapeDtypeStruct((N, K), jnp.int32),
            jax.ShapeDtypeStruct((N, C), jnp.float32),
            jax.ShapeDtypeStruct((N, C), jnp.float32),
        ],
    )(xb, xb, w1, wd, b2)


def _make_sc_aggregate(rows=N):
    nsc_ch = rows // (NW * SCH)
    mesh = plsc.VectorSubcoreMesh(core_axis_name="c", subcore_axis_name="s")

    @functools.partial(
        pl.kernel,
        mesh=mesh,
        out_type=jax.ShapeDtypeStruct((rows, C), jnp.float32),
        scratch_types=[
            pltpu.VMEM((2, NGA, RCH * K), jnp.int32),
            pltpu.VMEM((2, SCH * K, C), jnp.float32),
            pltpu.VMEM((2, SCH, C), jnp.float32),
            pltpu.VMEM((2, SCH, C), jnp.float32),
            pltpu.SemaphoreType.DMA,
            pltpu.SemaphoreType.DMA,
            pltpu.SemaphoreType.DMA,
            pltpu.SemaphoreType.DMA,
            pltpu.SemaphoreType.DMA,
            pltpu.SemaphoreType.DMA,
        ],
    )
    def agg(u_hbm, v_hbm, idx_hbm, out_hbm, idx_v, ug_v, v_v, o_v,
            si0, si1, sg0, sg1, so0, so1):
        wid = lax.axis_index("s") * 2 + lax.axis_index("c")
        base = wid * nsc_ch
        sem_i, sem_g, sem_o = (si0, si1), (sg0, sg1), (so0, so1)

        def fetch_idx(sc, p):
            g0 = pl.multiple_of((base + sc) * NGA, NGA)
            pltpu.async_copy(idx_hbm.at[pl.ds(g0, NGA)], idx_v.at[p],
                             sem_i[p])

        def wait_idx(p):
            pltpu.make_async_copy(idx_hbm.at[pl.ds(0, NGA)], idx_v.at[p],
                                  sem_i[p]).wait()

        def issue_main(sc, p):
            row0 = pl.multiple_of((base + sc) * SCH, SCH)
            for g in range(NGA):
                pltpu.async_copy(u_hbm.at[idx_v.at[p, g]],
                                 ug_v.at[p, pl.ds(g * RCH * K, RCH * K)],
                                 sem_g[p])
            pltpu.async_copy(v_hbm.at[pl.ds(row0, SCH)], v_v.at[p], sem_g[p])

        def wait_main(p):
            pltpu.make_async_copy(u_hbm.at[pl.ds(0, SCH * K)], ug_v.at[p],
                                  sem_g[p]).wait()
            pltpu.make_async_copy(v_hbm.at[pl.ds(0, SCH)], v_v.at[p],
                                  sem_g[p]).wait()

        def wait_out(p):
            pltpu.make_async_copy(o_v.at[p], out_hbm.at[pl.ds(0, SCH)],
                                  sem_o[p]).wait()

        def compute_store(sc, p):
            def row_body(r, carry, p=p):
                accs = [v_v[p, r, pl.ds(j * 16, 16)] for j in range(C // 16)]
                vvs = list(accs)
                for k in range(K):
                    for j in range(C // 16):
                        s = ug_v[p, r * K + k, pl.ds(j * 16, 16)] + vvs[j]
                        h = jnp.maximum(s, 0.2 * s)
                        accs[j] = h if k == 0 else accs[j] + h
                for j in range(C // 16):
                    o_v[p, r, pl.ds(j * 16, 16)] = accs[j] * (1.0 / K)
                return carry

            lax.fori_loop(0, SCH, row_body, 0)
            row0 = pl.multiple_of((base + sc) * SCH, SCH)
            pltpu.async_copy(o_v.at[p], out_hbm.at[pl.ds(row0, SCH)],
                             sem_o[p])

        fetch_idx(0, 0)
        wait_idx(0)
        issue_main(0, 0)
        if nsc_ch > 1:
            fetch_idx(1, 1)

        def pair_body(ii, carry):
            for pp in range(2):
                sc = ii * 2 + pp

                @pl.when(sc + 1 < nsc_ch)
                def _(pp=pp, sc=sc):
                    wait_idx(1 - pp)
                    issue_main(sc + 1, 1 - pp)

                wait_main(pp)

                @pl.when(sc + 2 < nsc_ch)
                def _(pp=pp, sc=sc):
                    fetch_idx(sc + 2, pp)

                @pl.when(sc >= 2)
                def _(pp=pp):
                    wait_out(pp)

                compute_store(sc, pp)
            return carry

        lax.fori_loop(0, nsc_ch // 2, pair_body, 0)
        wait_out(0)
        if nsc_ch > 1:
            wait_out(1)

    return agg


def kernel(x, W, b):
    w1 = W[:, :C]
    wd = W[:, C:] - w1
    b2 = b.reshape(1, C)
    outs = []
    for c0 in range(0, B, CHUNK):
        us, vs, ids = [], [], []
        for bi in range(c0, c0 + CHUNK):
            xb = lax.slice_in_dim(x, bi, bi + 1, axis=0)
            idx, u, v = _topk_uv(xb, w1, wd, b2)
            us.append(u); vs.append(v)
            ids.append(idx if CHUNK == 1 else idx + (bi - c0) * N)
        u_t = jnp.concatenate(us) if CHUNK > 1 else us[0]
        v_t = jnp.concatenate(vs) if CHUNK > 1 else vs[0]
        i_t = jnp.concatenate(ids) if CHUNK > 1 else ids[0]
        outs.append(_sc_agg(u_t, v_t,
                            i_t.reshape((CHUNK * N * K) // (RCH * K),
                                        RCH * K)))
    o = jnp.concatenate(outs) if len(outs) > 1 else outs[0]
    return o.reshape(B, N, C).transpose(0, 2, 1)


_sc_agg = _make_sc_aggregate(CHUNK * N)
